# Initial kernel scaffold; baseline (speedup 1.0000x reference)
import jax
import jax.numpy as jnp
from jax.experimental import pallas as pl


def kernel(boxes, scores):
    raise NotImplementedError("write your pallas kernel here")


# speedup vs baseline: 19.2333x; 19.2333x over previous
import jax
import jax.numpy as jnp
from jax import lax
from jax.experimental import pallas as pl
from jax.experimental.pallas import tpu as pltpu
from jax.experimental.pallas import tpu_sc as plsc

N = 5000
NP = 5120
NSUB = 16
CH = NP // NSUB
VPC = CH // 16
IOU_T = 0.3
SCORE_T = 0.05


def _iou_sup(lx1, ly1, lx2, ly2, la, x1, y1, x2, y2, a):
    xx1 = jnp.maximum(x1, lx1)
    yy1 = jnp.maximum(y1, ly1)
    xx2 = jnp.minimum(x2, lx2)
    yy2 = jnp.minimum(y2, ly2)
    w = jnp.maximum(0.0, xx2 - xx1 + 1.0)
    h = jnp.maximum(0.0, yy2 - yy1 + 1.0)
    inter = w * h
    iou = inter / (la + a - inter)
    return iou > IOU_T


def _nms_body(x1h, y1h, x2h, y2h, ah, vh, outh,
              x1v, y1v, x2v, y2v, av, pending, keep, lead, shared):
    sid = lax.axis_index("s")
    base = sid * CH
    lanes = lax.broadcasted_iota(jnp.int32, (16,), 0)

    pltpu.sync_copy(x1h, x1v)
    pltpu.sync_copy(y1h, y1v)
    pltpu.sync_copy(x2h, x2v)
    pltpu.sync_copy(y2h, y2v)
    pltpu.sync_copy(ah, av)
    pltpu.sync_copy(vh.at[pl.ds(base, CH)], pending)

    def zero_body(w, _):
        keep[pl.ds(w * 16, 16)] = jnp.zeros((16,), jnp.int32)
        return 0

    lax.fori_loop(0, VPC, zero_body, 0)

    def boxes_at(off):
        return (x1v[pl.ds(off, 16)], y1v[pl.ds(off, 16)],
                x2v[pl.ds(off, 16)], y2v[pl.ds(off, 16)], av[pl.ds(off, 16)])

    dnums = lax.GatherDimensionNumbers(
        offset_dims=(), collapsed_slice_dims=(0,), start_index_map=(0,))

    def leader_coords(off, lane):
        def pick(vec):
            return lax.gather(vec, lane[:, None], dnums, (1,),
                              mode=lax.GatherScatterMode.PROMISE_IN_BOUNDS)
        return tuple(pick(vec) for vec in boxes_at(off))

    def suppress_chunk(coords, vmin):
        lx1, ly1, lx2, ly2, la = coords
        for w in range(VPC):
            sup = _iou_sup(lx1, ly1, lx2, ly2, la, *boxes_at(base + w * 16))
            pv = pending[pl.ds(w * 16, 16)]
            newpv = jnp.where(sup, 0, pv)
            pending[pl.ds(w * 16, 16)] = jnp.where(w > vmin, newpv, pv)

    def resolve():
        def vloop(v, _):
            def kloop(k, _):
                pvec = pending[pl.ds(v * 16, 16)] != 0
                is_leader = jnp.any(jnp.logical_and(pvec, lanes == k))

                @pl.when(is_leader)
                def _():
                    lane = jnp.full((16,), 0, jnp.int32) + k
                    coords = leader_coords(base + v * 16, lane)
                    keep[pl.ds(v * 16, 16)] = jnp.where(
                        lanes == k, 1, keep[pl.ds(v * 16, 16)])
                    sup = _iou_sup(*coords, *boxes_at(base + v * 16))
                    pv = pending[pl.ds(v * 16, 16)]
                    keep_lane = jnp.logical_or(lanes <= k,
                                               jnp.logical_not(sup))
                    pending[pl.ds(v * 16, 16)] = jnp.where(keep_lane, pv, 0)
                    suppress_chunk(coords, v)

                return 0

            lax.fori_loop(0, 16, kloop, 0)
            return 0

        lax.fori_loop(0, VPC, vloop, 0)

    def apply_leaders(c):
        pltpu.sync_copy(shared.at[c], lead)

        def vloop(kv, _):
            def kloop(k, _):
                lvec = lead[pl.ds(kv * 16, 16)] != 0
                is_leader = jnp.any(jnp.logical_and(lvec, lanes == k))

                @pl.when(is_leader)
                def _():
                    lane = jnp.full((16,), 0, jnp.int32) + k
                    coords = leader_coords(c * CH + kv * 16, lane)
                    suppress_chunk(coords, -1)

                return 0

            lax.fori_loop(0, 16, kloop, 0)
            return 0

        lax.fori_loop(0, VPC, vloop, 0)

    def phase(c, _):
        @pl.when(sid == c)
        def _():
            resolve()
            pltpu.sync_copy(keep, shared.at[c])

        plsc.subcore_barrier()

        @pl.when(sid > c)
        def _():
            apply_leaders(c)

        return 0

    lax.fori_loop(0, NSUB, phase, 0)
    pltpu.sync_copy(keep, outh.at[pl.ds(base, CH)])


_mesh = plsc.VectorSubcoreMesh(
    core_axis_name="c", subcore_axis_name="s", num_cores=1)

_nms = pl.kernel(
    _nms_body,
    out_type=jax.ShapeDtypeStruct((NP,), jnp.int32),
    mesh=_mesh,
    scratch_types=[
        pltpu.VMEM((NP,), jnp.float32),
        pltpu.VMEM((NP,), jnp.float32),
        pltpu.VMEM((NP,), jnp.float32),
        pltpu.VMEM((NP,), jnp.float32),
        pltpu.VMEM((NP,), jnp.float32),
        pltpu.VMEM((CH,), jnp.int32),
        pltpu.VMEM((CH,), jnp.int32),
        pltpu.VMEM((CH,), jnp.int32),
        pltpu.VMEM_SHARED((NSUB, CH), jnp.int32),
    ],
    compiler_params=pltpu.CompilerParams(needs_layout_passes=False),
)


def kernel(boxes, scores):
    valid = scores >= SCORE_T
    sort_key = jnp.where(valid, scores, -jnp.inf)
    order = jnp.argsort(-sort_key)
    b = jnp.take(boxes, order, axis=0)
    s = jnp.take(scores, order, axis=0)
    v = jnp.take(valid, order, axis=0)

    x1, y1, x2, y2 = b[:, 0], b[:, 1], b[:, 2], b[:, 3]
    area = (x2 - x1 + 1.0) * (y2 - y1 + 1.0)

    pad = NP - N
    x1p = jnp.pad(x1, (0, pad))
    y1p = jnp.pad(y1, (0, pad))
    x2p = jnp.pad(x2, (0, pad))
    y2p = jnp.pad(y2, (0, pad))
    ap = jnp.pad(area, (0, pad), constant_values=1.0)
    vp = jnp.pad(v.astype(jnp.int32), (0, pad))

    keep_i = _nms(x1p, y1p, x2p, y2p, ap, vp)
    keep = keep_i[:N] > 0

    dets = jnp.concatenate([b, s[:, None]], axis=1)
    return jnp.where(keep[:, None], dets, 0.0)
